# Initial kernel scaffold; baseline (speedup 1.0000x reference)
import jax
import jax.numpy as jnp
from jax import lax
from jax.experimental import pallas as pl
from jax.experimental.pallas import tpu as pltpu
from jax.experimental.pallas import tpu_sc as plsc

B = 4096
N = 10
LW = 20
D = 32
SEG = N + 1
ROWS = SEG * LW
CHUNK = 112
PADROWS = 2 * CHUNK
NC, NS = 2, 16
NW = NC * NS
BPW = B // NW
VL = 16


def _sc_body(idx_hbm, table_hbm, out_hbm, idx_v, rows_v, out_v, sem_a, sem_b):
    wid = lax.axis_index("s") * NC + lax.axis_index("c")
    base = wid * BPW

    pltpu.sync_copy(idx_hbm.at[pl.ds(base, BPW)], idx_v)

    sems = (sem_a, sem_b)

    def issue(b, buf):
        pltpu.async_copy(table_hbm.at[idx_v.at[b, 0]],
                         rows_v.at[buf, pl.ds(0, CHUNK)], sems[buf])
        pltpu.async_copy(table_hbm.at[idx_v.at[b, 1]],
                         rows_v.at[buf, pl.ds(CHUNK, CHUNK)], sems[buf])

    def wait(b, buf):
        for c in range(2):
            pltpu.make_async_copy(table_hbm.at[idx_v.at[b, c]],
                                  rows_v.at[buf, pl.ds(c * CHUNK, CHUNK)],
                                  sems[buf]).wait()

    zeros = jnp.zeros((VL,), jnp.float32)

    def compute(b, buf):
        for s in range(SEG):
            acc = [zeros] * (6 if s == 0 else 4)
            for j in range(LW):
                r = s * LW + j
                v0 = rows_v[buf, r, pl.ds(0, VL)]
                v1 = rows_v[buf, r, pl.ds(VL, VL)]
                acc[0] = acc[0] + v0
                acc[1] = acc[1] + v1
                acc[2] = acc[2] + jnp.where(v0 != 0.0, 1.0, 0.0)
                acc[3] = acc[3] + jnp.where(v1 != 0.0, 1.0, 0.0)
                if s == 0:
                    acc[4] = acc[4] + v0 * v0
                    acc[5] = acc[5] + v1 * v1
            out_v[s, b, pl.ds(0, VL)] = acc[0] / acc[2]
            out_v[s, b, pl.ds(VL, VL)] = acc[1] / acc[3]
            if s == 0:
                out_v[11, b, pl.ds(0, VL)] = acc[0]
                out_v[11, b, pl.ds(VL, VL)] = acc[1]
                out_v[12, b, pl.ds(0, VL)] = acc[4]
                out_v[12, b, pl.ds(VL, VL)] = acc[5]

    issue(0, 0)

    @pl.loop(0, BPW, step=2)
    def _(i):
        for k in range(2):
            b = i + k
            wait(b, k)
            if k == 0:
                issue(b + 1, 1)
            else:
                @pl.when(b + 1 < BPW)
                def _():
                    issue(b + 1, 0)
            compute(b, k)

    for s in range(13):
        pltpu.sync_copy(out_v.at[s], out_hbm.at[s, pl.ds(base, BPW)])


def _tc_finish(sc_ref, words_ref, marg_ref, out_ref):
    i = pl.program_id(0)
    c = sc_ref[0]
    s1 = sc_ref[11]
    ssq = sc_ref[12]
    cnt2 = jnp.sum((words_ref[...] != 0).astype(jnp.float32), axis=1,
                   keepdims=True)
    cnorm = jnp.sum(c * c, axis=1, keepdims=True)
    cdot = jnp.sum(c * s1, axis=1, keepdims=True)
    s2 = jnp.sum(ssq, axis=1, keepdims=True)
    pos = 0.5 * (cnt2 * (cnorm + 1e-9) - 2.0 * cdot + s2) / cnt2

    marg = marg_ref[...]
    acc = jnp.zeros_like(pos)
    for n in range(N):
        cn = sc_ref[1 + n]
        d2 = jnp.sum((c - cn) ** 2, axis=1, keepdims=True)
        t = jnp.maximum(marg[:, n:n + 1] - jnp.sqrt(d2 + 1e-9), 0.0)
        acc += t * t
    neg = 0.5 * acc / float(N)

    @pl.when(i == 0)
    def _():
        out_ref[0, 0] = 0.0
    out_ref[0, 0] += jnp.sum(pos + neg)

    @pl.when(i == pl.num_programs(0) - 1)
    def _():
        out_ref[0, 0] = out_ref[0, 0] / float(B)


@jax.jit
def kernel(syn_words, neg_words, margins, table):
    syn_words = syn_words.astype(jnp.int32)
    neg_words = neg_words.astype(jnp.int32)
    idx = jnp.concatenate(
        [syn_words[:, None, :], neg_words], axis=1).reshape(B, ROWS)
    idx = jnp.concatenate(
        [idx, jnp.zeros((B, PADROWS - ROWS), jnp.int32)], axis=1)
    idx = idx.reshape(B, 2, CHUNK)

    mesh = plsc.VectorSubcoreMesh(
        core_axis_name="c", subcore_axis_name="s",
        num_cores=NC, num_subcores=NS)
    sc_out = pl.kernel(
        _sc_body,
        out_type=jax.ShapeDtypeStruct((13, B, D), jnp.float32),
        mesh=mesh,
        scratch_types=[
            pltpu.VMEM((BPW, 2, CHUNK), jnp.int32),
            pltpu.VMEM((2, PADROWS, D), jnp.float32),
            pltpu.VMEM((13, BPW, D), jnp.float32),
            pltpu.SemaphoreType.DMA,
            pltpu.SemaphoreType.DMA,
        ],
    )(idx, table)

    R = 512
    loss = pl.pallas_call(
        _tc_finish,
        grid=(B // R,),
        in_specs=[
            pl.BlockSpec((13, R, D), lambda i: (0, i, 0)),
            pl.BlockSpec((R, LW), lambda i: (i, 0)),
            pl.BlockSpec((R, N), lambda i: (i, 0)),
        ],
        out_specs=pl.BlockSpec((1, 1), lambda i: (0, 0)),
        out_shape=jax.ShapeDtypeStruct((1, 1), jnp.float32),
    )(sc_out, syn_words, margins)
    return loss[0, 0]


# speedup vs baseline: 3.0656x; 3.0656x over previous
import jax
import jax.numpy as jnp
from jax import lax
from jax.experimental import pallas as pl
from jax.experimental.pallas import tpu as pltpu
from jax.experimental.pallas import tpu_sc as plsc

B = 4096
N = 10
LW = 20
D = 32
SEG = N + 1
ROWS = SEG * LW
CHUNK = 112
PADROWS = 2 * CHUNK
NC, NS = 2, 16
NW = NC * NS
BPW = B // NW
VL = 16


def _sc_body(idx_hbm, table_hbm, out_hbm, idx_v, rows_v, out_v, sem_a, sem_b):
    wid = lax.axis_index("s") * NC + lax.axis_index("c")
    base = wid * BPW

    pltpu.sync_copy(idx_hbm.at[pl.ds(base, BPW)], idx_v)

    sems = (sem_a, sem_b)

    def issue(b, buf):
        pltpu.async_copy(table_hbm.at[idx_v.at[b, 0]],
                         rows_v.at[buf, pl.ds(0, CHUNK)], sems[buf])
        pltpu.async_copy(table_hbm.at[idx_v.at[b, 1]],
                         rows_v.at[buf, pl.ds(CHUNK, CHUNK)], sems[buf])

    def wait(b, buf):
        for c in range(2):
            pltpu.make_async_copy(table_hbm.at[idx_v.at[b, c]],
                                  rows_v.at[buf, pl.ds(c * CHUNK, CHUNK)],
                                  sems[buf]).wait()

    zeros = jnp.zeros((VL,), jnp.float32)

    def compute(b, buf):
        for s in range(SEG):
            acc = [zeros] * (6 if s == 0 else 4)
            for j in range(LW):
                r = s * LW + j
                v0 = rows_v[buf, r, pl.ds(0, VL)]
                v1 = rows_v[buf, r, pl.ds(VL, VL)]
                acc[0] = acc[0] + v0
                acc[1] = acc[1] + v1
                acc[2] = acc[2] + jnp.where(v0 != 0.0, 1.0, 0.0)
                acc[3] = acc[3] + jnp.where(v1 != 0.0, 1.0, 0.0)
                if s == 0:
                    acc[4] = acc[4] + v0 * v0
                    acc[5] = acc[5] + v1 * v1
            out_v[s, b, pl.ds(0, VL)] = acc[0] / acc[2]
            out_v[s, b, pl.ds(VL, VL)] = acc[1] / acc[3]
            if s == 0:
                out_v[11, b, pl.ds(0, VL)] = acc[0]
                out_v[11, b, pl.ds(VL, VL)] = acc[1]
                out_v[12, b, pl.ds(0, VL)] = acc[4]
                out_v[12, b, pl.ds(VL, VL)] = acc[5]

    issue(0, 0)

    @pl.loop(0, BPW, step=2)
    def _(i):
        for k in range(2):
            b = i + k
            wait(b, k)
            if k == 0:
                issue(b + 1, 1)
            else:
                @pl.when(b + 1 < BPW)
                def _():
                    issue(b + 1, 0)
            compute(b, k)

    for s in range(13):
        pltpu.sync_copy(out_v.at[s], out_hbm.at[s, pl.ds(base, BPW)])


def _tc_finish(sc_ref, words_ref, marg_ref, out_ref):
    i = pl.program_id(0)
    c = sc_ref[0]
    s1 = sc_ref[11]
    ssq = sc_ref[12]
    cnt2 = jnp.sum((words_ref[...] != 0).astype(jnp.float32), axis=1,
                   keepdims=True)
    cnorm = jnp.sum(c * c, axis=1, keepdims=True)
    cdot = jnp.sum(c * s1, axis=1, keepdims=True)
    s2 = jnp.sum(ssq, axis=1, keepdims=True)
    pos = 0.5 * (cnt2 * (cnorm + 1e-9) - 2.0 * cdot + s2) / cnt2

    marg = marg_ref[...]
    acc = jnp.zeros_like(pos)
    for n in range(N):
        cn = sc_ref[1 + n]
        d2 = jnp.sum((c - cn) ** 2, axis=1, keepdims=True)
        t = jnp.maximum(marg[:, n:n + 1] - jnp.sqrt(d2 + 1e-9), 0.0)
        acc += t * t
    neg = 0.5 * acc / float(N)

    @pl.when(i == 0)
    def _():
        out_ref[...] = jnp.zeros((1, 1), jnp.float32)
    out_ref[...] += jnp.sum(pos + neg, keepdims=True)

    @pl.when(i == pl.num_programs(0) - 1)
    def _():
        out_ref[...] = out_ref[...] / float(B)


@jax.jit
def kernel(syn_words, neg_words, margins, table):
    syn_words = syn_words.astype(jnp.int32)
    neg_words = neg_words.astype(jnp.int32)
    idx = jnp.concatenate(
        [syn_words[:, None, :], neg_words], axis=1).reshape(B, ROWS)
    idx = jnp.concatenate(
        [idx, jnp.zeros((B, PADROWS - ROWS), jnp.int32)], axis=1)
    idx = idx.reshape(B, 2, CHUNK)

    mesh = plsc.VectorSubcoreMesh(
        core_axis_name="c", subcore_axis_name="s",
        num_cores=NC, num_subcores=NS)
    sc_out = pl.kernel(
        _sc_body,
        out_type=jax.ShapeDtypeStruct((13, B, D), jnp.float32),
        mesh=mesh,
        scratch_types=[
            pltpu.VMEM((BPW, 2, CHUNK), jnp.int32),
            pltpu.VMEM((2, PADROWS, D), jnp.float32),
            pltpu.VMEM((13, BPW, D), jnp.float32),
            pltpu.SemaphoreType.DMA,
            pltpu.SemaphoreType.DMA,
        ],
        compiler_params=pltpu.CompilerParams(use_tc_tiling_on_sc=False),
    )(idx, table)

    R = 512
    loss = pl.pallas_call(
        _tc_finish,
        grid=(B // R,),
        in_specs=[
            pl.BlockSpec((13, R, D), lambda i: (0, i, 0)),
            pl.BlockSpec((R, LW), lambda i: (i, 0)),
            pl.BlockSpec((R, N), lambda i: (i, 0)),
        ],
        out_specs=pl.BlockSpec((1, 1), lambda i: (0, 0)),
        out_shape=jax.ShapeDtypeStruct((1, 1), jnp.float32),
    )(sc_out, syn_words, margins)
    return loss[0, 0]
